# Initial kernel scaffold; baseline (speedup 1.0000x reference)
import jax
import jax.numpy as jnp
from jax.experimental import pallas as pl


def kernel(item_ids, item_emb_weight):
    raise NotImplementedError("write your pallas kernel here")


# speedup vs baseline: 1.4997x; 1.4997x over previous
import functools

import jax
import jax.numpy as jnp
from jax import lax
from jax.experimental import pallas as pl
from jax.experimental.pallas import tpu as pltpu
from jax.experimental.pallas import tpu_sc as plsc

_INFO = plsc.get_sparse_core_info()
_NC, _NS = _INFO.num_cores, _INFO.num_subcores
_NW = _NC * _NS


@functools.lru_cache(maxsize=None)
def _build_gather(n, v, d):
    per_w = n // _NW
    chunk = 3200
    nchunk = per_w // chunk
    assert per_w % chunk == 0 and per_w % 8 == 0

    mesh = plsc.VectorSubcoreMesh(core_axis_name="c", subcore_axis_name="s")

    @functools.partial(
        pl.kernel,
        mesh=mesh,
        out_type=jax.ShapeDtypeStruct((n, d), jnp.float32),
        scratch_types=[
            pltpu.VMEM((per_w,), jnp.int32),
            pltpu.VMEM((chunk, d), jnp.float32),
            pltpu.SemaphoreType.DMA,
        ],
        compiler_params=pltpu.CompilerParams(use_tc_tiling_on_sc=False),
    )
    def gather_kernel(idx_hbm, table_hbm, out_hbm, idx_v, rows_v, gsem):
        wid = lax.axis_index("s") * _NC + lax.axis_index("c")
        base = pl.multiple_of(wid * per_w, per_w)
        pltpu.sync_copy(idx_hbm.at[pl.ds(base, per_w)], idx_v)

        def body(i, carry):
            off = pl.multiple_of(i * chunk, chunk)
            pltpu.async_copy(
                table_hbm.at[idx_v.at[pl.ds(off, chunk)]], rows_v, gsem
            ).wait()
            pltpu.sync_copy(rows_v, out_hbm.at[pl.ds(base + off, chunk)])
            return carry

        lax.fori_loop(0, nchunk, body, 0)

    return gather_kernel


def kernel(item_ids, item_emb_weight):
    b, h = item_ids.shape
    v, d = item_emb_weight.shape
    ids_flat = item_ids.reshape(b * h).astype(jnp.int32)
    fn = _build_gather(b * h, v, d)
    out = fn(ids_flat, item_emb_weight)
    return out.reshape(b, h, d)


# speedup vs baseline: 1.5004x; 1.0005x over previous
import functools

import jax
import jax.numpy as jnp
from jax import lax
from jax.experimental import pallas as pl
from jax.experimental.pallas import tpu as pltpu
from jax.experimental.pallas import tpu_sc as plsc

_INFO = plsc.get_sparse_core_info()
_NC, _NS = _INFO.num_cores, _INFO.num_subcores
_NW = _NC * _NS


@functools.lru_cache(maxsize=None)
def _build_gather(n, v, d):
    per_w = n // _NW
    chunk = 800
    nbuf = 4
    nchunk = per_w // chunk
    assert per_w % chunk == 0 and chunk % 8 == 0

    mesh = plsc.VectorSubcoreMesh(core_axis_name="c", subcore_axis_name="s")

    @functools.partial(
        pl.kernel,
        mesh=mesh,
        out_type=jax.ShapeDtypeStruct((n, d), jnp.float32),
        scratch_types=[
            pltpu.VMEM((per_w,), jnp.int32),
            pltpu.VMEM((nbuf, chunk, d), jnp.float32),
            [pltpu.SemaphoreType.DMA] * nbuf,
            [pltpu.SemaphoreType.DMA] * nbuf,
        ],
        compiler_params=pltpu.CompilerParams(use_tc_tiling_on_sc=False),
    )
    def gather_kernel(idx_hbm, table_hbm, out_hbm, idx_v, rows_v, gsem, ssem):
        wid = lax.axis_index("s") * _NC + lax.axis_index("c")
        base = pl.multiple_of(wid * per_w, per_w)
        pltpu.sync_copy(idx_hbm.at[pl.ds(base, per_w)], idx_v)

        def start_gather(i, b):
            pltpu.async_copy(
                table_hbm.at[idx_v.at[pl.ds(i * chunk, chunk)]],
                rows_v.at[b],
                gsem[b],
            )

        for b in range(nbuf):
            start_gather(b, b)
        for i in range(nchunk):
            b = i % nbuf
            pltpu.make_async_copy(
                table_hbm.at[idx_v.at[pl.ds(i * chunk, chunk)]],
                rows_v.at[b],
                gsem[b],
            ).wait()
            pltpu.async_copy(
                rows_v.at[b], out_hbm.at[pl.ds(base + i * chunk, chunk)], ssem[b]
            )
            if i + nbuf < nchunk:
                pltpu.make_async_copy(
                    rows_v.at[b],
                    out_hbm.at[pl.ds(base + i * chunk, chunk)],
                    ssem[b],
                ).wait()
                start_gather(i + nbuf, b)
        for i in range(nchunk - nbuf, nchunk):
            b = i % nbuf
            pltpu.make_async_copy(
                rows_v.at[b], out_hbm.at[pl.ds(base + i * chunk, chunk)], ssem[b]
            ).wait()

    return gather_kernel


def kernel(item_ids, item_emb_weight):
    b, h = item_ids.shape
    v, d = item_emb_weight.shape
    ids_flat = item_ids.reshape(b * h).astype(jnp.int32)
    fn = _build_gather(b * h, v, d)
    out = fn(ids_flat, item_emb_weight)
    return out.reshape(b, h, d)
